# baseline (speedup 1.0000x reference)
import functools

import jax
import jax.numpy as jnp
from jax import lax
from jax.experimental import pallas as pl
from jax.experimental.pallas import tpu as pltpu
from jax.experimental.pallas import tpu_sc as plsc

_B = 16384
_D = 64
_NC = 1
_NS = 16
_NW = _NC * _NS
_B_PER_W = _B // _NW
_HALF = _B_PER_W // 2
_G = 16

_mesh = plsc.VectorSubcoreMesh(
    core_axis_name="c", subcore_axis_name="s", num_cores=_NC
)


@functools.partial(
    pl.kernel,
    mesh=_mesh,
    out_type=jax.ShapeDtypeStruct((_B, _D), jnp.float32),
    scratch_types=[
        pltpu.VMEM((_B_PER_W,), jnp.int32),
        pltpu.VMEM((_HALF, _D), jnp.float32),
        pltpu.SemaphoreType.DMA,
    ],
)
def _gather(idx_hbm, table_hbm, out_hbm, idx_v, rows_v, sem):
    wid = lax.axis_index("s")
    base = wid * _B_PER_W
    pltpu.sync_copy(idx_hbm.at[pl.ds(base, _B_PER_W)], idx_v)

    def half(h, _):
        def group(g, _):
            ivec = idx_v[pl.ds(h * _HALF + g * _G, _G)]
            for j in range(_G):
                pltpu.async_copy(
                    table_hbm.at[pl.ds(ivec[j], 1)],
                    rows_v.at[pl.ds(g * _G + j, 1)],
                    sem,
                )
            return ()

        lax.fori_loop(0, _HALF // _G, group, (), unroll=False)
        pltpu.make_async_copy(
            table_hbm.at[pl.ds(0, _HALF)], rows_v, sem
        ).wait()
        pltpu.sync_copy(rows_v, out_hbm.at[pl.ds(base + h * _HALF, _HALF)])
        return ()

    lax.fori_loop(0, 2, half, (), unroll=False)


def kernel(idx, latents):
    return _gather(idx.astype(jnp.int32), latents)


# speedup vs baseline: 1.4172x; 1.4172x over previous
import functools

import jax
import jax.numpy as jnp
from jax import lax
from jax.experimental import pallas as pl
from jax.experimental.pallas import tpu as pltpu
from jax.experimental.pallas import tpu_sc as plsc

_V = 1_000_000
_B = 16384
_D = 64
_NC = 2
_NS = 16
_NW = _NC * _NS
_B_PER_W = _B // _NW
_PASS = _B_PER_W // 2
_C = 4
_NSUB = _PASS // _C
_TAIL = (_V // 128) * 128
_TMAX = _TAIL - 128

_mesh = plsc.VectorSubcoreMesh(core_axis_name="c", subcore_axis_name="s")


def _iota16():
    return lax.iota(jnp.int32, 16)


def _splat(x):
    return jnp.broadcast_to(x, (16,))


@functools.partial(
    pl.kernel,
    mesh=_mesh,
    out_type=jax.ShapeDtypeStruct((_B, _D), jnp.float32),
    scratch_types=[
        pltpu.VMEM((_B_PER_W + 32,), jnp.int32),
        pltpu.VMEM((_C, _D, 128), jnp.float32),
        pltpu.VMEM((_C, _D, 128), jnp.float32),
        pltpu.VMEM((_PASS, _D), jnp.float32),
        pltpu.VMEM((_D, _D), jnp.float32),
        pltpu.SemaphoreType.DMA,
        pltpu.SemaphoreType.DMA,
    ],
    compiler_params=pltpu.CompilerParams(needs_layout_passes=False),
)
def _gather(idx_hbm, tab_hbm, tail_hbm, out_hbm, idx_v, sb_a, sb_b, rows_v,
            tail_v, sem_a, sem_b):
    wid = lax.axis_index("s") * _NC + lax.axis_index("c")
    base = wid * _B_PER_W
    pltpu.sync_copy(idx_hbm.at[pl.ds(base, _B_PER_W)], idx_v.at[pl.ds(0, _B_PER_W)])
    pltpu.sync_copy(tail_hbm, tail_v)

    def slab_start(r):
        return pl.multiple_of(
            jnp.minimum(lax.shift_right_logical(r, 7) * 128, _TMAX), 128
        )

    def fire(pbase, ss, sb, sem):
        ivec = idx_v[pl.ds(pbase + ss * _C, 16)]
        for u in range(_C):
            t = slab_start(ivec[u])
            pltpu.async_copy(tab_hbm.at[:, pl.ds(t, 128)], sb.at[u], sem)

    def drain(sb, sem):
        for u in range(_C):
            pltpu.make_async_copy(
                tab_hbm.at[:, pl.ds(0, 128)], sb.at[u], sem
            ).wait()

    def extract(ss, sb):
        pbase_l = 0
        ivec = idx_v[pl.ds(_pass_base[0] + ss * _C, 16)]
        for u in range(_C):
            r = ivec[u]
            t = slab_start(r)
            l = jnp.minimum(r - t, 127)
            row = ss * _C + u + pbase_l
            for b in range(_D // 16):
                vals = plsc.load_gather(
                    sb, [_splat(u), _iota16() + 16 * b, _splat(l)]
                )
                plsc.store_scatter(
                    rows_v, [_splat(row), _iota16() + 16 * b], vals
                )

            @pl.when(r >= _TAIL)
            def _():
                lt = r - _TAIL
                for b in range(_D // 16):
                    vals = plsc.load_gather(
                        tail_v, [_iota16() + 16 * b, _splat(lt)]
                    )
                    plsc.store_scatter(
                        rows_v, [_splat(row), _iota16() + 16 * b], vals
                    )

    _pass_base = [0]
    for h in range(2):
        pbase = h * _PASS
        _pass_base[0] = pbase
        fire(pbase, 0, sb_a, sem_a)

        def body(i, _):
            ss_a = 2 * i
            ss_b = 2 * i + 1
            fire(pbase, ss_b, sb_b, sem_b)
            drain(sb_a, sem_a)
            extract(ss_a, sb_a)

            @pl.when(i < _NSUB // 2 - 1)
            def _():
                fire(pbase, ss_a + 2, sb_a, sem_a)

            drain(sb_b, sem_b)
            extract(ss_b, sb_b)
            return ()

        lax.fori_loop(0, _NSUB // 2, body, (), unroll=False)
        pltpu.sync_copy(rows_v, out_hbm.at[pl.ds(base + pbase, _PASS)])


def kernel(idx, latents):
    idx32 = idx.astype(jnp.int32)
    table_t = latents.T
    tail_t = latents[_TAIL:, :].T
    return _gather(idx32, table_t, tail_t)


# speedup vs baseline: 1.6838x; 1.1881x over previous
import functools

import jax
import jax.numpy as jnp
from jax import lax
from jax.experimental import pallas as pl
from jax.experimental.pallas import tpu as pltpu
from jax.experimental.pallas import tpu_sc as plsc

_V = 1_000_000
_B = 16384
_D = 64
_NC = 2
_NS = 16
_NW = _NC * _NS
_B_PER_W = _B // _NW
_PASS = _B_PER_W // 2
_R = 8
_NG = _PASS // 16
_TAIL = (_V // 128) * 128
_TMAX = _TAIL - 128

_mesh = plsc.VectorSubcoreMesh(core_axis_name="c", subcore_axis_name="s")


def _iota16():
    return lax.iota(jnp.int32, 16)


def _splat(x):
    return jnp.broadcast_to(x, (16,))


@functools.partial(
    pl.kernel,
    mesh=_mesh,
    out_type=jax.ShapeDtypeStruct((_B, _D), jnp.float32),
    scratch_types=[
        pltpu.VMEM((_B_PER_W + 32,), jnp.int32),
        pltpu.VMEM((_R, _D, 128), jnp.float32),
        pltpu.VMEM((_PASS, _D), jnp.float32),
        pltpu.VMEM((_D, _D), jnp.float32),
        [pltpu.SemaphoreType.DMA] * _R,
    ],
    compiler_params=pltpu.CompilerParams(needs_layout_passes=False),
)
def _gather(idx_hbm, tab_hbm, tail_hbm, out_hbm, idx_v, sb, rows_v, tail_v,
            sems):
    wid = lax.axis_index("s") * _NC + lax.axis_index("c")
    base = wid * _B_PER_W
    pltpu.sync_copy(
        idx_hbm.at[pl.ds(base, _B_PER_W)], idx_v.at[pl.ds(0, _B_PER_W)]
    )
    pltpu.sync_copy(tail_hbm, tail_v)

    def slab_start(r):
        return pl.multiple_of(
            jnp.minimum(lax.shift_right_logical(r, 7) * 128, _TMAX), 128
        )

    def fire(r, slot):
        pltpu.async_copy(
            tab_hbm.at[:, pl.ds(slab_start(r), 128)], sb.at[slot], sems[slot]
        )

    def wait(slot):
        pltpu.make_async_copy(
            tab_hbm.at[:, pl.ds(0, 128)], sb.at[slot], sems[slot]
        ).wait()

    def extract(r, slot, row):
        l = jnp.minimum(r - slab_start(r), 127)
        for b in range(_D // 16):
            vals = plsc.load_gather(
                sb, [_splat(slot), _iota16() + 16 * b, _splat(l)]
            )
            plsc.store_scatter(
                rows_v, [_splat(row), _iota16() + 16 * b], vals
            )

        @pl.when(r >= _TAIL)
        def _():
            lt = r - _TAIL
            for b in range(_D // 16):
                vals = plsc.load_gather(
                    tail_v, [_iota16() + 16 * b, _splat(lt)]
                )
                plsc.store_scatter(
                    rows_v, [_splat(row), _iota16() + 16 * b], vals
                )

    for h in range(2):
        pbase = h * _PASS
        ivec0 = idx_v[pl.ds(pbase, 16)]
        for u in range(_R):
            fire(ivec0[u], u)

        def body(g, _):
            goff = pbase + g * 16
            ivec = idx_v[pl.ds(goff, 16)]
            ivecn = idx_v[pl.ds(goff + 16, 16)]
            for u in range(16):
                slot = u % _R
                wait(slot)
                extract(ivec[u], slot, g * 16 + u)
                if u < _R:
                    fire(ivec[u + _R], slot)
                else:
                    @pl.when(g < _NG - 1)
                    def _():
                        fire(ivecn[u - _R], slot)
            return ()

        lax.fori_loop(0, _NG, body, (), unroll=False)
        pltpu.sync_copy(rows_v, out_hbm.at[pl.ds(base + pbase, _PASS)])


def kernel(idx, latents):
    idx32 = idx.astype(jnp.int32)
    table_t = latents.T
    tail_t = latents[_TAIL:, :].T
    return _gather(idx32, table_t, tail_t)
